# baseline (speedup 1.0000x reference)
import functools

import jax
import jax.numpy as jnp
from jax import lax
from jax.experimental import pallas as pl
from jax.experimental.pallas import tpu as pltpu
from jax.experimental.pallas import tpu_sc as plsc

B = 4096
NNZ = 256
M = 2048
L = 16

NC = 2
NS = 16
NW = NC * NS
ROWS = B // NW
G = 8
NG = ROWS // G
NQ = NNZ // L
Z = 4


def _body(idx_hbm, val_hbm, zero_hbm, out_hbm, idx_v, val_v, dense0, dense1,
          ssem, sem0, sem1, zsem0, zsem1):
    c = lax.axis_index("c")
    s = lax.axis_index("s")
    wid = s * NC + c
    base = wid * ROWS

    zeros16 = jnp.zeros((L,), jnp.float32)
    bufs = (dense0, dense1)
    sems = (sem0, sem1)
    zsems = (zsem0, zsem1)

    pltpu.async_copy(idx_hbm.at[pl.ds(base, ROWS)], idx_v, ssem)
    pltpu.async_copy(val_hbm.at[pl.ds(base, ROWS)], val_v, ssem)

    def zero_group(b):
        dense = bufs[b]

        @plsc.parallel_loop(0, G * M // L, unroll=16)
        def _zero(i):
            dense[i // (M // L), pl.ds((i % (M // L)) * L, L)] = zeros16

    def hybrid_zero(b):
        dense = bufs[b]
        pltpu.async_copy(zero_hbm, dense.at[pl.ds(0, Z)], zsems[b])

        @plsc.parallel_loop(0, (G - Z) * M // L, unroll=16)
        def _zero(i):
            dense[Z + i // (M // L), pl.ds((i % (M // L)) * L, L)] = zeros16

        pltpu.make_async_copy(zero_hbm, dense.at[pl.ds(0, Z)], zsems[b]).wait()

    def scatter_group(b, g0):
        dense = bufs[b]

        @plsc.parallel_loop(0, G, unroll=2)
        def _scatter_row(g):
            r = g0 + g
            gvec = jnp.full((L,), 0, jnp.int32) + g
            for q in range(NQ):
                idx16 = idx_v[r, pl.ds(q * L, L)]
                val16 = val_v[r, pl.ds(q * L, L)]
                plsc.addupdate_scatter(dense, [gvec, idx16], val16)

        pltpu.async_copy(
            dense, out_hbm.at[pl.ds(base + g0, G)], sems[b]
        )

    def wait_group(b, g0):
        pltpu.make_async_copy(
            bufs[b], out_hbm.at[pl.ds(base + g0, G)], sems[b]
        ).wait()

    zero_group(0)
    zero_group(1)
    pltpu.make_async_copy(
        idx_hbm.at[pl.ds(base, ROWS)], idx_v, ssem).wait()
    pltpu.make_async_copy(
        val_hbm.at[pl.ds(base, ROWS)], val_v, ssem).wait()

    scatter_group(0, 0)
    scatter_group(1, G)

    def pair(p, carry):
        g0 = 2 * p * G
        wait_group(0, g0 - 2 * G)
        hybrid_zero(0)
        scatter_group(0, g0)
        wait_group(1, g0 - G)
        hybrid_zero(1)
        scatter_group(1, g0 + G)
        return carry

    lax.fori_loop(1, NG // 2, pair, 0)
    wait_group(0, (NG - 2) * G)
    wait_group(1, (NG - 1) * G)


_sc_call = functools.partial(
    pl.kernel,
    mesh=plsc.VectorSubcoreMesh(core_axis_name="c", subcore_axis_name="s"),
    out_type=jax.ShapeDtypeStruct((B, M), jnp.float32),
    compiler_params=pltpu.CompilerParams(needs_layout_passes=False),
    scratch_types=[
        pltpu.VMEM((ROWS, NNZ), jnp.int32),
        pltpu.VMEM((ROWS, NNZ), jnp.float32),
        pltpu.VMEM((G, M), jnp.float32),
        pltpu.VMEM((G, M), jnp.float32),
        pltpu.SemaphoreType.DMA,
        pltpu.SemaphoreType.DMA,
        pltpu.SemaphoreType.DMA,
        pltpu.SemaphoreType.DMA,
        pltpu.SemaphoreType.DMA,
    ],
)(_body)


def kernel(indices, values):
    zero_src = jnp.zeros((Z, M), jnp.float32)
    return _sc_call(indices, values, zero_src)


# speedup vs baseline: 1.9729x; 1.9729x over previous
import functools

import jax
import jax.numpy as jnp
from jax import lax
from jax.experimental import pallas as pl
from jax.experimental.pallas import tpu as pltpu
from jax.experimental.pallas import tpu_sc as plsc

B = 4096
NNZ = 256
M = 2048
L = 16

NC = 2
NS = 16
NW = NC * NS
ROWS = B // NW
G = 8
NG = ROWS // G
NQ = NNZ // L


def _body(idx_hbm, val_hbm, out_hbm, idx_v, val_v, dense0, dense1,
          ssem, sem0, sem1):
    c = lax.axis_index("c")
    s = lax.axis_index("s")
    wid = s * NC + c
    base = wid * ROWS

    zeros16 = jnp.zeros((L,), jnp.float32)
    bufs = (dense0, dense1)
    sems = (sem0, sem1)

    pltpu.async_copy(idx_hbm.at[pl.ds(base, ROWS)], idx_v, ssem)
    pltpu.async_copy(val_hbm.at[pl.ds(base, ROWS)], val_v, ssem)

    def zero_group(b):
        dense = bufs[b]

        @plsc.parallel_loop(0, G * M // L, unroll=16)
        def _zero(i):
            dense[i // (M // L), pl.ds((i % (M // L)) * L, L)] = zeros16

    def scatter_group(b, g0):
        dense = bufs[b]

        @plsc.parallel_loop(0, G, unroll=4)
        def _scatter_row(g):
            r = g0 + g
            gvec = jnp.full((L,), 0, jnp.int32) + g
            for q in range(NQ):
                idx16 = idx_v[r, pl.ds(q * L, L)]
                val16 = val_v[r, pl.ds(q * L, L)]
                plsc.addupdate_scatter(dense, [gvec, idx16], val16)

        pltpu.async_copy(
            dense, out_hbm.at[pl.ds(base + g0, G)], sems[b]
        )

    def wait_group(b, g0):
        pltpu.make_async_copy(
            bufs[b], out_hbm.at[pl.ds(base + g0, G)], sems[b]
        ).wait()

    zero_group(0)
    zero_group(1)
    pltpu.make_async_copy(
        idx_hbm.at[pl.ds(base, ROWS)], idx_v, ssem).wait()
    pltpu.make_async_copy(
        val_hbm.at[pl.ds(base, ROWS)], val_v, ssem).wait()

    scatter_group(0, 0)
    scatter_group(1, G)

    def pair(p, carry):
        g0 = 2 * p * G
        wait_group(0, g0 - 2 * G)
        zero_group(0)
        scatter_group(0, g0)
        wait_group(1, g0 - G)
        zero_group(1)
        scatter_group(1, g0 + G)
        return carry

    lax.fori_loop(1, NG // 2, pair, 0)
    wait_group(0, (NG - 2) * G)
    wait_group(1, (NG - 1) * G)


_sc_call = functools.partial(
    pl.kernel,
    mesh=plsc.VectorSubcoreMesh(core_axis_name="c", subcore_axis_name="s"),
    out_type=jax.ShapeDtypeStruct((B, M), jnp.float32),
    compiler_params=pltpu.CompilerParams(needs_layout_passes=False),
    scratch_types=[
        pltpu.VMEM((ROWS, NNZ), jnp.int32),
        pltpu.VMEM((ROWS, NNZ), jnp.float32),
        pltpu.VMEM((G, M), jnp.float32),
        pltpu.VMEM((G, M), jnp.float32),
        pltpu.SemaphoreType.DMA,
        pltpu.SemaphoreType.DMA,
        pltpu.SemaphoreType.DMA,
    ],
)(_body)


def kernel(indices, values):
    return _sc_call(indices, values)
